# Initial kernel scaffold; baseline (speedup 1.0000x reference)
import jax
import jax.numpy as jnp
from jax.experimental import pallas as pl


def kernel(origin_input, direction_input, z_vals, viewdirs, weights, u):
    raise NotImplementedError("write your pallas kernel here")


# speedup vs baseline: 161.6942x; 161.6942x over previous
import jax
import jax.numpy as jnp
from jax import lax
from jax.experimental import pallas as pl


def _bitonic_stage(v, j, k, up=True):
    n = v.shape[1]
    i = lax.broadcasted_iota(jnp.int32, (1, n), 1)
    bitj = (i & j) == 0
    dirup = ((i & k) == 0) == up
    other = jnp.where(bitj, jnp.roll(v, -j, axis=1), jnp.roll(v, j, axis=1))
    takemin = bitj == dirup
    return jnp.where(takemin, jnp.minimum(v, other), jnp.maximum(v, other))


def _fine_samples_body(org_ref, dir_ref, z_ref, w_ref, u_ref,
                       zall_ref, px_ref, py_ref, pz_ref):
    z = z_ref[...]
    u = u_ref[...]
    R, Nc = z.shape
    S = u.shape[1]
    nb = Nc - 2
    nc = Nc - 1

    w = w_ref[:, 1:Nc - 1] + jnp.float32(1e-5)
    pdf = w / jnp.sum(w, axis=1, keepdims=True)

    jj = lax.broadcasted_iota(jnp.int32, (nb, nc), 0)
    kk = lax.broadcasted_iota(jnp.int32, (nb, nc), 1)
    tri = (jj < kk).astype(jnp.float32)
    cdf = lax.dot_general(pdf, tri, (((1,), (0,)), ((), ())),
                          precision=lax.Precision.HIGHEST,
                          preferred_element_type=jnp.float32)

    mid = jnp.float32(0.5) * (z[:, 1:] + z[:, :-1])

    c3 = cdf[:, None, :]
    m3 = mid[:, None, :]
    u3 = u[:, :, None]
    mask = c3 <= u3
    neg = jnp.float32(-jnp.inf)
    pos = jnp.float32(jnp.inf)
    cdfb = jnp.max(jnp.where(mask, c3, neg), axis=2)
    binb = jnp.max(jnp.where(mask, m3, neg), axis=2)
    cdfa = jnp.min(jnp.where(mask, pos, c3), axis=2)
    bina = jnp.min(jnp.where(mask, pos, m3), axis=2)
    has_above = cdfa < pos
    cdfa = jnp.where(has_above, cdfa, cdfb)
    bina = jnp.where(has_above, bina, binb)

    denom = cdfa - cdfb
    denom = jnp.where(denom < jnp.float32(1e-5), jnp.float32(1.0), denom)
    t = (u - cdfb) / denom
    zs = binb + t * (bina - binb)

    k = 2
    while k <= S:
        j = k // 2
        while j >= 1:
            zs = _bitonic_stage(zs, j, k, up=False)
            j //= 2
        k *= 2

    pad = jnp.full((R, S - Nc), pos, dtype=jnp.float32)
    buf = jnp.concatenate([z, pad, zs], axis=1)
    j = S
    while j >= 1:
        buf = _bitonic_stage(buf, j, 2 * S)
        j //= 2
    zall = buf[:, :Nc + S]

    zall_ref[...] = zall
    px_ref[...] = org_ref[:, 0:1] + dir_ref[:, 0:1] * zall
    py_ref[...] = org_ref[:, 1:2] + dir_ref[:, 1:2] * zall
    pz_ref[...] = org_ref[:, 2:3] + dir_ref[:, 2:3] * zall


def _run(origin_input, direction_input, z_vals, viewdirs, weights, u,
         interpret=False):
    B, Nc = z_vals.shape
    S = u.shape[1]
    T = Nc + S
    R = 8
    grid = (B // R,)

    row3 = pl.BlockSpec((R, 3), lambda i: (i, 0))
    out_shape = [jax.ShapeDtypeStruct((B, T), jnp.float32)] * 4
    zall, px, py, pz = pl.pallas_call(
        _fine_samples_body,
        grid=grid,
        in_specs=[
            row3,
            row3,
            pl.BlockSpec((R, Nc), lambda i: (i, 0)),
            pl.BlockSpec((R, Nc), lambda i: (i, 0)),
            pl.BlockSpec((R, S), lambda i: (i, 0)),
        ],
        out_specs=[
            pl.BlockSpec((R, T), lambda i: (i, 0)),
            pl.BlockSpec((R, T), lambda i: (i, 0)),
            pl.BlockSpec((R, T), lambda i: (i, 0)),
            pl.BlockSpec((R, T), lambda i: (i, 0)),
        ],
        out_shape=out_shape,
        interpret=interpret,
    )(origin_input, direction_input, z_vals, weights, u)

    pts = jnp.stack([px, py, pz], axis=-1)
    return (pts, viewdirs, zall)


def kernel(origin_input, direction_input, z_vals, viewdirs, weights, u):
    return _run(origin_input, direction_input, z_vals, viewdirs, weights, u)


# speedup vs baseline: 268.5928x; 1.6611x over previous
import jax
import jax.numpy as jnp
from jax import lax
from jax.experimental import pallas as pl


def _bitonic_stage(v, j, k, up=True):
    n = v.shape[1]
    i = lax.broadcasted_iota(jnp.int32, (1, n), 1)
    bitj = (i & j) == 0
    dirup = ((i & k) == 0) == up
    other = jnp.where(bitj, jnp.roll(v, -j, axis=1), jnp.roll(v, j, axis=1))
    takemin = bitj == dirup
    return jnp.where(takemin, jnp.minimum(v, other), jnp.maximum(v, other))


def _fine_samples_body(org_ref, dir_ref, z_ref, w_ref, u_ref,
                       zall_ref, px_ref, py_ref, pz_ref):
    z = z_ref[...]
    u = u_ref[...]
    R, Nc = z.shape
    S = u.shape[1]
    nb = Nc - 2
    nc = Nc - 1

    w = w_ref[:, 1:Nc - 1] + jnp.float32(1e-5)
    pdf = w / jnp.sum(w, axis=1, keepdims=True)

    jj = lax.broadcasted_iota(jnp.int32, (nb, nc), 0)
    kk = lax.broadcasted_iota(jnp.int32, (nb, nc), 1)
    tri = (jj < kk).astype(jnp.float32)
    cdf = lax.dot_general(pdf, tri, (((1,), (0,)), ((), ())),
                          precision=lax.Precision.HIGHEST,
                          preferred_element_type=jnp.float32)

    mid = jnp.float32(0.5) * (z[:, 1:] + z[:, :-1])

    cdfn = jnp.concatenate([cdf[:, 1:], cdf[:, nc - 1:nc]], axis=1)
    midn = jnp.concatenate([mid[:, 1:], mid[:, nc - 1:nc]], axis=1)
    c3 = cdf[:, None, :]
    m3 = mid[:, None, :]
    cn3 = cdfn[:, None, :]
    mn3 = midn[:, None, :]
    u3 = u[:, :, None]
    mask = c3 <= u3
    neg = jnp.float32(-jnp.inf)
    pos = jnp.float32(jnp.inf)
    cdfb = jnp.max(jnp.where(mask, c3, neg), axis=2)
    binb = jnp.max(jnp.where(mask, m3, neg), axis=2)
    cdfa = jnp.max(jnp.where(mask, cn3, neg), axis=2)
    bina = jnp.max(jnp.where(mask, mn3, neg), axis=2)

    denom = cdfa - cdfb
    denom = jnp.where(denom < jnp.float32(1e-5), jnp.float32(1.0), denom)
    t = (u - cdfb) / denom
    zs = binb + t * (bina - binb)

    k = 2
    while k <= S:
        j = k // 2
        while j >= 1:
            zs = _bitonic_stage(zs, j, k, up=False)
            j //= 2
        k *= 2

    pad = jnp.full((R, S - Nc), pos, dtype=jnp.float32)
    buf = jnp.concatenate([z, pad, zs], axis=1)
    j = S
    while j >= 1:
        buf = _bitonic_stage(buf, j, 2 * S)
        j //= 2
    zall = buf[:, :Nc + S]

    zall_ref[...] = zall
    px_ref[...] = org_ref[:, 0:1] + dir_ref[:, 0:1] * zall
    py_ref[...] = org_ref[:, 1:2] + dir_ref[:, 1:2] * zall
    pz_ref[...] = org_ref[:, 2:3] + dir_ref[:, 2:3] * zall


def _run(origin_input, direction_input, z_vals, viewdirs, weights, u,
         interpret=False):
    B, Nc = z_vals.shape
    S = u.shape[1]
    T = Nc + S
    R = 32
    grid = (B // R,)

    row3 = pl.BlockSpec((R, 3), lambda i: (i, 0))
    out_shape = [jax.ShapeDtypeStruct((B, T), jnp.float32)] * 4
    zall, px, py, pz = pl.pallas_call(
        _fine_samples_body,
        grid=grid,
        in_specs=[
            row3,
            row3,
            pl.BlockSpec((R, Nc), lambda i: (i, 0)),
            pl.BlockSpec((R, Nc), lambda i: (i, 0)),
            pl.BlockSpec((R, S), lambda i: (i, 0)),
        ],
        out_specs=[
            pl.BlockSpec((R, T), lambda i: (i, 0)),
            pl.BlockSpec((R, T), lambda i: (i, 0)),
            pl.BlockSpec((R, T), lambda i: (i, 0)),
            pl.BlockSpec((R, T), lambda i: (i, 0)),
        ],
        out_shape=out_shape,
        interpret=interpret,
    )(origin_input, direction_input, z_vals, weights, u)

    pts = jnp.stack([px, py, pz], axis=-1)
    return (pts, viewdirs, zall)


def kernel(origin_input, direction_input, z_vals, viewdirs, weights, u):
    return _run(origin_input, direction_input, z_vals, viewdirs, weights, u)


# speedup vs baseline: 1076.3899x; 4.0075x over previous
import jax
import jax.numpy as jnp
from jax import lax
from jax.experimental import pallas as pl


def _bitonic_stage(v, j, k, up=True):
    n = v.shape[1]
    i = lax.broadcasted_iota(jnp.int32, (1, n), 1)
    bitj = (i & j) == 0
    dirup = ((i & k) == 0) == up
    other = jnp.where(bitj, jnp.roll(v, -j, axis=1), jnp.roll(v, j, axis=1))
    takemin = bitj == dirup
    return jnp.where(takemin, jnp.minimum(v, other), jnp.maximum(v, other))


def _fine_samples_body(org_ref, dir_ref, z_ref, w_ref, u_ref,
                       zall_ref, px_ref, py_ref, pz_ref):
    z = z_ref[...]
    u = u_ref[...]
    R, Nc = z.shape
    S = u.shape[1]
    nb = Nc - 2
    nc = Nc - 1

    w = w_ref[:, 1:Nc - 1] + jnp.float32(1e-5)
    pdf = w / jnp.sum(w, axis=1, keepdims=True)

    jj = lax.broadcasted_iota(jnp.int32, (nb, nc), 0)
    kk = lax.broadcasted_iota(jnp.int32, (nb, nc), 1)
    tri = (jj < kk).astype(jnp.float32)
    cdf = lax.dot_general(pdf, tri, (((1,), (0,)), ((), ())),
                          precision=lax.Precision.HIGHEST,
                          preferred_element_type=jnp.float32)

    mid = jnp.float32(0.5) * (z[:, 1:] + z[:, :-1])

    cdfn = jnp.concatenate([cdf[:, 1:], cdf[:, nc - 1:nc]], axis=1)
    midn = jnp.concatenate([mid[:, 1:], mid[:, nc - 1:nc]], axis=1)
    c3 = cdf[:, :, None]
    m3 = mid[:, :, None]
    cn3 = cdfn[:, :, None]
    mn3 = midn[:, :, None]
    u3 = u[:, None, :]
    mask = c3 <= u3
    neg = jnp.float32(-jnp.inf)
    pos = jnp.float32(jnp.inf)
    cdfb = jnp.max(jnp.where(mask, c3, neg), axis=1)
    binb = jnp.max(jnp.where(mask, m3, neg), axis=1)
    cdfa = jnp.max(jnp.where(mask, cn3, neg), axis=1)
    bina = jnp.max(jnp.where(mask, mn3, neg), axis=1)

    denom = cdfa - cdfb
    denom = jnp.where(denom < jnp.float32(1e-5), jnp.float32(1.0), denom)
    t = (u - cdfb) / denom
    zs = binb + t * (bina - binb)

    k = 2
    while k <= S:
        j = k // 2
        while j >= 1:
            zs = _bitonic_stage(zs, j, k, up=False)
            j //= 2
        k *= 2

    pad = jnp.full((R, S - Nc), pos, dtype=jnp.float32)
    buf = jnp.concatenate([z, pad, zs], axis=1)
    j = S
    while j >= 1:
        buf = _bitonic_stage(buf, j, 2 * S)
        j //= 2
    zall = buf[:, :Nc + S]

    zall_ref[...] = zall
    px_ref[...] = org_ref[:, 0:1] + dir_ref[:, 0:1] * zall
    py_ref[...] = org_ref[:, 1:2] + dir_ref[:, 1:2] * zall
    pz_ref[...] = org_ref[:, 2:3] + dir_ref[:, 2:3] * zall


def _run(origin_input, direction_input, z_vals, viewdirs, weights, u,
         interpret=False):
    B, Nc = z_vals.shape
    S = u.shape[1]
    T = Nc + S
    R = 256
    grid = (B // R,)

    row3 = pl.BlockSpec((R, 3), lambda i: (i, 0))
    out_shape = [jax.ShapeDtypeStruct((B, T), jnp.float32)] * 4
    zall, px, py, pz = pl.pallas_call(
        _fine_samples_body,
        grid=grid,
        in_specs=[
            row3,
            row3,
            pl.BlockSpec((R, Nc), lambda i: (i, 0)),
            pl.BlockSpec((R, Nc), lambda i: (i, 0)),
            pl.BlockSpec((R, S), lambda i: (i, 0)),
        ],
        out_specs=[
            pl.BlockSpec((R, T), lambda i: (i, 0)),
            pl.BlockSpec((R, T), lambda i: (i, 0)),
            pl.BlockSpec((R, T), lambda i: (i, 0)),
            pl.BlockSpec((R, T), lambda i: (i, 0)),
        ],
        out_shape=out_shape,
        interpret=interpret,
    )(origin_input, direction_input, z_vals, weights, u)

    pts = jnp.stack([px, py, pz], axis=-1)
    return (pts, viewdirs, zall)


def kernel(origin_input, direction_input, z_vals, viewdirs, weights, u):
    return _run(origin_input, direction_input, z_vals, viewdirs, weights, u)


# speedup vs baseline: 1340.2177x; 1.2451x over previous
import functools

import jax
import jax.numpy as jnp
from jax import lax
from jax.experimental import pallas as pl
from jax.experimental.pallas import tpu as pltpu
from jax.experimental.pallas import tpu_sc as plsc


def _sc_sample_body(nchunks, ch, rpw,
                    z_hbm, w_hbm, u_hbm, out_hbm,
                    zbuf, wbuf, ubuf, cdfbuf, outbuf):
    L = 16
    wid = lax.axis_index("s") * 2 + lax.axis_index("c")
    iota = lax.iota(jnp.int32, L)
    Nc = 128
    nbins = Nc - 2
    last = Nc - 2

    def chunk_body(g, _):
        base = wid * rpw + g * ch
        pltpu.sync_copy(z_hbm.at[pl.ds(base, ch)], zbuf)
        pltpu.sync_copy(w_hbm.at[pl.ds(base, ch)], wbuf)
        pltpu.sync_copy(u_hbm.at[pl.ds(base, ch)], ubuf)

        def ray_body(r, _):
            rfull = jnp.full((L,), r, jnp.int32)

            acc = jnp.zeros((L,), jnp.float32)
            for c in range(8):
                j = iota + (16 * c)
                valid = j < nbins
                wg = plsc.load_gather(wbuf, [rfull, jnp.minimum(j + 1, 127)])
                acc = acc + jnp.where(valid, wg, jnp.float32(0.0))
            wsum = jnp.sum(acc) + jnp.float32(nbins * 1e-5)
            wsum_b = jnp.full((L,), wsum, jnp.float32)

            carry = jnp.zeros((L,), jnp.float32)
            for c in range(8):
                j = iota + (16 * c)
                valid = j < nbins
                wg = plsc.load_gather(wbuf, [rfull, jnp.minimum(j + 1, 127)])
                pc = jnp.where(valid, (wg + jnp.float32(1e-5)) / wsum_b,
                               jnp.float32(0.0))
                cum = plsc.cumsum(pc) + carry
                cdfbuf[pl.ds(16 * c, 16)] = cum
                carry = jnp.full((L,), jnp.max(cum), jnp.float32)

            for q in range(16):
                uq = plsc.load_gather(ubuf, [rfull, iota + (16 * q)])
                lo = jnp.zeros((L,), jnp.int32)
                for step in (64, 32, 16, 8, 4, 2, 1):
                    probe = lo + step
                    cv = plsc.load_gather(cdfbuf, [probe - 1])
                    take = jnp.logical_and(cv <= uq, probe <= last)
                    lo = jnp.where(take, probe, lo)
                above = jnp.minimum(lo + 1, last)
                cdfb = plsc.load_gather(cdfbuf, [jnp.maximum(lo - 1, 0)])
                cdfb = jnp.where(lo == 0, jnp.float32(0.0), cdfb)
                cdfa = plsc.load_gather(cdfbuf, [above - 1])
                zb0 = plsc.load_gather(zbuf, [rfull, lo])
                zb1 = plsc.load_gather(zbuf, [rfull, lo + 1])
                za0 = plsc.load_gather(zbuf, [rfull, above])
                za1 = plsc.load_gather(zbuf, [rfull, above + 1])
                binb = jnp.float32(0.5) * (zb0 + zb1)
                bina = jnp.float32(0.5) * (za0 + za1)
                denom = cdfa - cdfb
                denom = jnp.where(denom < jnp.float32(1e-5), jnp.float32(1.0),
                                  denom)
                t = (uq - cdfb) / denom
                zs = binb + t * (bina - binb)
                outbuf[r, pl.ds(16 * q, 16)] = zs
            return 0

        lax.fori_loop(0, ch, ray_body, 0)
        pltpu.sync_copy(outbuf, out_hbm.at[pl.ds(base, ch)])
        return 0

    lax.fori_loop(0, nchunks, chunk_body, 0)


def _sc_sample(z_vals, weights, u):
    B, Nc = z_vals.shape
    S = u.shape[1]
    NW = 32
    rpw = B // NW
    ch = 8
    nchunks = rpw // ch
    mesh = plsc.VectorSubcoreMesh(core_axis_name="c", subcore_axis_name="s")
    body = functools.partial(_sc_sample_body, nchunks, ch, rpw)
    fn = pl.kernel(
        body,
        mesh=mesh,
        compiler_params=pltpu.CompilerParams(needs_layout_passes=False),
        out_type=jax.ShapeDtypeStruct((B, S), jnp.float32),
        scratch_types=[
            pltpu.VMEM((ch, Nc), jnp.float32),
            pltpu.VMEM((ch, Nc), jnp.float32),
            pltpu.VMEM((ch, S), jnp.float32),
            pltpu.VMEM((Nc,), jnp.float32),
            pltpu.VMEM((ch, S), jnp.float32),
        ],
    )
    return fn(z_vals, weights, u)


def _bitonic_stage(v, j, k, up=True):
    n = v.shape[1]
    i = lax.broadcasted_iota(jnp.int32, (1, n), 1)
    bitj = (i & j) == 0
    dirup = ((i & k) == 0) == up
    other = jnp.where(bitj, jnp.roll(v, -j, axis=1), jnp.roll(v, j, axis=1))
    takemin = bitj == dirup
    return jnp.where(takemin, jnp.minimum(v, other), jnp.maximum(v, other))


def _tc_sort_body(org_ref, dir_ref, z_ref, zs_ref,
                  zall_ref, px_ref, py_ref, pz_ref):
    z = z_ref[...]
    zs = zs_ref[...]
    R, Nc = z.shape
    S = zs.shape[1]
    pos = jnp.float32(jnp.inf)

    k = 2
    while k <= S:
        j = k // 2
        while j >= 1:
            zs = _bitonic_stage(zs, j, k, up=False)
            j //= 2
        k *= 2

    pad = jnp.full((R, S - Nc), pos, dtype=jnp.float32)
    buf = jnp.concatenate([z, pad, zs], axis=1)
    j = S
    while j >= 1:
        buf = _bitonic_stage(buf, j, 2 * S)
        j //= 2
    zall = buf[:, :Nc + S]

    zall_ref[...] = zall
    px_ref[...] = org_ref[:, 0:1] + dir_ref[:, 0:1] * zall
    py_ref[...] = org_ref[:, 1:2] + dir_ref[:, 1:2] * zall
    pz_ref[...] = org_ref[:, 2:3] + dir_ref[:, 2:3] * zall


def _tc_sort(origin_input, direction_input, z_vals, zs):
    B, Nc = z_vals.shape
    S = zs.shape[1]
    T = Nc + S
    R = 256
    grid = (B // R,)
    row3 = pl.BlockSpec((R, 3), lambda i: (i, 0))
    out_shape = [jax.ShapeDtypeStruct((B, T), jnp.float32)] * 4
    return pl.pallas_call(
        _tc_sort_body,
        grid=grid,
        in_specs=[
            row3,
            row3,
            pl.BlockSpec((R, Nc), lambda i: (i, 0)),
            pl.BlockSpec((R, S), lambda i: (i, 0)),
        ],
        out_specs=[pl.BlockSpec((R, T), lambda i: (i, 0))] * 4,
        out_shape=out_shape,
    )(origin_input, direction_input, z_vals, zs)


def kernel(origin_input, direction_input, z_vals, viewdirs, weights, u):
    zs = _sc_sample(z_vals, weights, u)
    zall, px, py, pz = _tc_sort(origin_input, direction_input, z_vals, zs)
    pts = jnp.stack([px, py, pz], axis=-1)
    return (pts, viewdirs, zall)
